# baseline (speedup 1.0000x reference)
import functools

import jax
import jax.numpy as jnp
from jax import lax
from jax.experimental import pallas as pl
from jax.experimental.pallas import tpu as pltpu
from jax.experimental.pallas import tpu_sc as plsc

B = 4096
L = 200
EMB = 64
NC = 2
NS = 16
NW = NC * NS
RPW = B // NW
C0, C1 = 104, 96

_mesh = plsc.VectorSubcoreMesh(core_axis_name="c", subcore_axis_name="s")


@functools.partial(
    pl.kernel,
    out_type=jax.ShapeDtypeStruct((B,), jnp.float32),
    mesh=_mesh,
    compiler_params=pltpu.CompilerParams(
        needs_layout_passes=False, use_tc_tiling_on_sc=False),
    scratch_types=[
        pltpu.VMEM((RPW * L,), jnp.int32),
        pltpu.VMEM((L, EMB), jnp.float32),
        pltpu.VMEM((RPW,), jnp.int32),
        pltpu.VMEM((EMB,), jnp.float32),
        pltpu.VMEM((16,), jnp.float32),
        pltpu.VMEM((RPW,), jnp.float32),
        pltpu.SemaphoreType.DMA,
        pltpu.SemaphoreType.DMA,
    ],
)
def _fasttext_sc(data_hbm, len_hbm, table_hbm, w_hbm, b_hbm, out_hbm,
                 idx_v, rows_v, len_v, w_v, b_v, out_v, sem0, sem1):
    wid = lax.axis_index("s") * NC + lax.axis_index("c")
    base = wid * RPW

    pltpu.sync_copy(data_hbm.at[pl.ds(base * L, RPW * L)], idx_v)
    pltpu.sync_copy(len_hbm.at[pl.ds(base, RPW)], len_v)
    pltpu.sync_copy(w_hbm, w_v)
    pltpu.sync_copy(b_hbm, b_v.at[pl.ds(0, 1)])

    w0 = w_v[pl.ds(0, 16)]
    w1 = w_v[pl.ds(16, 16)]
    w2 = w_v[pl.ds(32, 16)]
    w3 = w_v[pl.ds(48, 16)]
    bias = b_v[pl.ds(0, 16)][0]
    lane = lax.iota(jnp.int32, 16)
    zero = jnp.zeros((16,), jnp.float32)

    def group_body(g, _):
        zvec = zero
        for j in range(16):
            i = g * 16 + j
            cp0 = pltpu.async_copy(
                table_hbm.at[idx_v.at[pl.ds(i * L, C0)]],
                rows_v.at[pl.ds(0, C0)], sem0)
            cp1 = pltpu.async_copy(
                table_hbm.at[idx_v.at[pl.ds(i * L + C0, C1)]],
                rows_v.at[pl.ds(C0, C1)], sem1)
            cp0.wait()
            cp1.wait()

            def acc_body(t, carry):
                a0, a1, a2, a3 = carry
                a0 = a0 + rows_v[t, pl.ds(0, 16)]
                a1 = a1 + rows_v[t, pl.ds(16, 16)]
                a2 = a2 + rows_v[t, pl.ds(32, 16)]
                a3 = a3 + rows_v[t, pl.ds(48, 16)]
                return (a0, a1, a2, a3)

            a0, a1, a2, a3 = lax.fori_loop(
                0, L, acc_body, (zero, zero, zero, zero))
            p = a0 * w0 + a1 * w1 + a2 * w2 + a3 * w3
            s = jnp.sum(p)
            zvec = jnp.where(lane == j, s, zvec)
        lvec = len_v[pl.ds(g * 16, 16)].astype(jnp.float32)
        zvec = zvec / lvec + bias
        sig = 1.0 / (1.0 + jnp.exp(-zvec))
        out_v[pl.ds(g * 16, 16)] = sig
        return 0

    lax.fori_loop(0, RPW // 16, group_body, 0)
    pltpu.sync_copy(out_v, out_hbm.at[pl.ds(base, RPW)])


def kernel(data, length, emb_table, W, b):
    return _fasttext_sc(data.reshape(-1), length, emb_table,
                        W.reshape(-1), b)
